# baseline (speedup 1.0000x reference)
import functools

import jax
import jax.numpy as jnp
from jax.experimental import pallas as pl
from jax.experimental.pallas import tpu as pltpu

N = 2048
H = 256
B = 1024
_F32 = jnp.float32


def _elu(x):
    return jnp.where(x > 0, x, jnp.exp(jnp.minimum(x, 0.0)) - 1.0)


def _dot(a, b):
    return jnp.dot(a, b, preferred_element_type=_F32)


def _dotT(a, b):
    return jax.lax.dot_general(a, b, (((0,), (0,)), ((), ())),
                               preferred_element_type=_F32)


def _bdot(a, b):
    return jnp.dot(a.astype(jnp.bfloat16), b.astype(jnp.bfloat16),
                   preferred_element_type=_F32)


def _stage1_body(xk_ref, xp_ref, wk_ref, bk_ref, wp_ref, bp_ref,
                 wsk_ref, wsp_ref, b2_ref, out_ref):
    ak = _elu(_bdot(xk_ref[:], wk_ref[:]) + bk_ref[:])
    ap = _elu(_bdot(xp_ref[:], wp_ref[:]) + bp_ref[:])
    out_ref[:] = _bdot(ak, wsk_ref[:]) + _bdot(ap, wsp_ref[:]) + b2_ref[:]


def _stage2_body(node_ref, idx_ref, wt_ref, bt_ref, wg1_ref, bg1_ref,
                 wg2_ref, bg2_ref, out_ref):
    node = node_ref[:]
    res = node
    ne = _elu(node)
    trans = jnp.tanh(_dot(ne, wt_ref[:]) + bt_ref[:])
    inv = jax.lax.rsqrt(jnp.sum(trans * trans, axis=1, keepdims=True))
    that = trans * inv
    y = _dot(ne, wg1_ref[:])
    res = res + _dot(that, _dotT(that, y)) + bg1_ref[:]
    ne = _elu(res)
    y = _dot(ne, wg2_ref[:])
    res = res + _dot(that, _dotT(that, y)) + bg2_ref[:]
    iota = jax.lax.broadcasted_iota(jnp.int32, (B, N), 1)
    onehot = (iota == idx_ref[:]).astype(jnp.bfloat16)
    out_ref[:] = _bdot(onehot, res)


def _pair_body(ga_ref, gv_ref, wgt_ref, wgb_ref, bg_ref, wp_ref, bp_ref,
               out_ref):
    ga = _elu(ga_ref[:])
    gv = _elu(gv_ref[:])
    h = _elu(_dot(ga, wgt_ref[:]) + _dot(gv, wgb_ref[:]) + bg_ref[:])
    out_ref[:] = _dot(h, wp_ref[:]) + bp_ref[:]


def _branch(xk, xp, idx, wk, bk, wp, bp, wsk, wsp, b2, wt, bt,
            wg1, bg1, wg2, bg2):
    grid = 8
    blk = N // grid
    pk = xp.shape[1]
    node = pl.pallas_call(
        _stage1_body,
        grid=(grid,),
        in_specs=[
            pl.BlockSpec((blk, xk.shape[1]), lambda i: (i, 0)),
            pl.BlockSpec((blk, pk), lambda i: (i, 0)),
            pl.BlockSpec((xk.shape[1], H), lambda i: (0, 0)),
            pl.BlockSpec((1, H), lambda i: (0, 0)),
            pl.BlockSpec((pk, H), lambda i: (0, 0)),
            pl.BlockSpec((1, H), lambda i: (0, 0)),
            pl.BlockSpec((H, H), lambda i: (0, 0)),
            pl.BlockSpec((H, H), lambda i: (0, 0)),
            pl.BlockSpec((1, H), lambda i: (0, 0)),
        ],
        out_specs=pl.BlockSpec((blk, H), lambda i: (i, 0)),
        out_shape=jax.ShapeDtypeStruct((N, H), _F32),
    )(xk, xp, wk, bk, wp, bp, wsk, wsp, b2)
    return pl.pallas_call(
        _stage2_body,
        out_shape=jax.ShapeDtypeStruct((B, H), _F32),
    )(node, idx, wt, bt, wg1, bg1, wg2, bg2)


def kernel(antibody_graph_node_kmer_ft, antibody_graph_node_pssm_ft,
           virus_graph_node_kmer_ft, virus_graph_node_pssm_ft,
           antibody_idx, virus_idx, W_ab_k, b_ab_k, W_ab_p, b_ab_p,
           W_v_k, b_v_k, W_v_p, b_v_p, conv_w, conv_b, W_share, b_share,
           W_g1, b_g1, W_g2, b_g2, W_ab_t, b_ab_t, W_v_t, b_v_t,
           W_glob, b_glob, W_pred, b_pred):
    ws2 = (conv_w[None, :, None] * W_share[:, None, :]).reshape(2 * H, H)
    wsk, wsp = ws2[:H], ws2[H:]
    b2 = (b_share + conv_b * jnp.sum(W_share, axis=0)).reshape(1, H)

    row = lambda b: b.reshape(1, -1)
    ai = antibody_idx.astype(jnp.int32).reshape(B, 1)
    vi = virus_idx.astype(jnp.int32).reshape(B, 1)
    ga = _branch(antibody_graph_node_kmer_ft, antibody_graph_node_pssm_ft,
                 ai, W_ab_k, row(b_ab_k), W_ab_p, row(b_ab_p), wsk, wsp,
                 b2, W_ab_t, row(b_ab_t), W_g1, row(b_g1), W_g2, row(b_g2))
    gv = _branch(virus_graph_node_kmer_ft, virus_graph_node_pssm_ft,
                 vi, W_v_k, row(b_v_k), W_v_p, row(b_v_p), wsk, wsp,
                 b2, W_v_t, row(b_v_t), W_g1, row(b_g1), W_g2, row(b_g2))

    out = pl.pallas_call(
        _pair_body,
        out_shape=jax.ShapeDtypeStruct((B, 1), _F32),
    )(ga, gv, W_glob[:H], W_glob[H:], row(b_glob), W_pred, row(b_pred))
    return out


# speedup vs baseline: 1.0908x; 1.0908x over previous
import jax
import jax.numpy as jnp
from jax.experimental import pallas as pl
from jax.experimental.pallas import tpu as pltpu

N = 2048
H = 256
B = 1024
GRID = 8
BLK = N // GRID
_F32 = jnp.float32


def _elu(x):
    return jnp.where(x > 0, x, jnp.exp(jnp.minimum(x, 0.0)) - 1.0)


def _dot(a, b):
    return jnp.dot(a, b, preferred_element_type=_F32)


def _dotT(a, b):
    return jax.lax.dot_general(a, b, (((0,), (0,)), ((), ())),
                               preferred_element_type=_F32)


def _mega_body(xk_ab, xp_ab, xk_v, xp_v, ai, vi,
               wk_ab, bk_ab, wp_ab, bp_ab, wk_v, bk_v, wp_v, bp_v,
               wsk, wsp, b2, wt_ab, bt_ab, wt_v, bt_v,
               wg1, bg1, wg2, bg2, wgt, wgb, bg, wpr, bpr,
               out_ref, node_ab, node_v, ga, gv):
    i = pl.program_id(0)

    def stage1(xk_ref, xp_ref, wk, bk, wp, bp, node_ref, tile):
        ak = _elu(_dot(xk_ref[:], wk[:]) + bk[:])
        ap = _elu(_dot(xp_ref[:], wp[:]) + bp[:])
        node_ref[pl.ds(tile * BLK, BLK), :] = (
            _dot(ak, wsk[:]) + _dot(ap, wsp[:]) + b2[:])

    def stage2(node_ref, idx_ref, wt, bt, g_ref):
        node = node_ref[:]
        res = node
        ne = _elu(node)
        trans = jnp.tanh(_dot(ne, wt[:]) + bt[:])
        inv = jax.lax.rsqrt(jnp.sum(trans * trans, axis=1, keepdims=True))
        that = trans * inv
        y = _dot(ne, wg1[:])
        res = res + _dot(that, _dotT(that, y)) + bg1[:]
        ne = _elu(res)
        y = _dot(ne, wg2[:])
        res = res + _dot(that, _dotT(that, y)) + bg2[:]
        iota = jax.lax.broadcasted_iota(jnp.int32, (B, N), 1)
        onehot = (iota == idx_ref[:]).astype(_F32)
        g_ref[:] = _dot(onehot, res)

    @pl.when(i < GRID)
    def _():
        stage1(xk_ab, xp_ab, wk_ab, bk_ab, wp_ab, bp_ab, node_ab, i)

    @pl.when(i == GRID)
    def _():
        stage2(node_ab, ai, wt_ab, bt_ab, ga)

    @pl.when((i > GRID) & (i < 2 * GRID + 1))
    def _():
        stage1(xk_v, xp_v, wk_v, bk_v, wp_v, bp_v, node_v, i - GRID - 1)

    @pl.when(i == 2 * GRID + 1)
    def _():
        stage2(node_v, vi, wt_v, bt_v, gv)

    @pl.when(i == 2 * GRID + 2)
    def _():
        h = _elu(_dot(_elu(ga[:]), wgt[:]) + _dot(_elu(gv[:]), wgb[:])
                 + bg[:])
        out_ref[:] = _dot(h, wpr[:]) + bpr[:]


def kernel(antibody_graph_node_kmer_ft, antibody_graph_node_pssm_ft,
           virus_graph_node_kmer_ft, virus_graph_node_pssm_ft,
           antibody_idx, virus_idx, W_ab_k, b_ab_k, W_ab_p, b_ab_p,
           W_v_k, b_v_k, W_v_p, b_v_p, conv_w, conv_b, W_share, b_share,
           W_g1, b_g1, W_g2, b_g2, W_ab_t, b_ab_t, W_v_t, b_v_t,
           W_glob, b_glob, W_pred, b_pred):
    ws2 = (conv_w[None, :, None] * W_share[:, None, :]).reshape(2 * H, H)
    wsk, wsp = ws2[:H], ws2[H:]
    b2 = (b_share + conv_b * jnp.sum(W_share, axis=0)).reshape(1, H)

    row = lambda b: b.reshape(1, -1)
    ai = antibody_idx.astype(jnp.int32).reshape(B, 1)
    vi = virus_idx.astype(jnp.int32).reshape(B, 1)

    kp = antibody_graph_node_kmer_ft.shape[1]
    pa = antibody_graph_node_pssm_ft.shape[1]
    pv = virus_graph_node_pssm_ft.shape[1]

    def tile_ab(i):
        return (jnp.minimum(i, GRID - 1), 0)

    def tile_v(i):
        return (jnp.clip(i - GRID - 1, 0, GRID - 1), 0)

    def full(a):
        return pl.BlockSpec(a.shape, lambda i: (0,) * a.ndim)

    weights = [W_ab_k, row(b_ab_k), W_ab_p, row(b_ab_p),
               W_v_k, row(b_v_k), W_v_p, row(b_v_p),
               wsk, wsp, b2, W_ab_t, row(b_ab_t), W_v_t, row(b_v_t),
               W_g1, row(b_g1), W_g2, row(b_g2),
               W_glob[:H], W_glob[H:], row(b_glob), W_pred, row(b_pred)]

    out = pl.pallas_call(
        _mega_body,
        grid=(2 * GRID + 3,),
        in_specs=[
            pl.BlockSpec((BLK, kp), tile_ab),
            pl.BlockSpec((BLK, pa), tile_ab),
            pl.BlockSpec((BLK, kp), tile_v),
            pl.BlockSpec((BLK, pv), tile_v),
            full(ai),
            full(vi),
        ] + [full(w) for w in weights],
        out_specs=pl.BlockSpec((B, 1), lambda i: (0, 0)),
        out_shape=jax.ShapeDtypeStruct((B, 1), _F32),
        scratch_shapes=[
            pltpu.VMEM((N, H), _F32),
            pltpu.VMEM((N, H), _F32),
            pltpu.VMEM((B, H), _F32),
            pltpu.VMEM((B, H), _F32),
        ],
    )(antibody_graph_node_kmer_ft, antibody_graph_node_pssm_ft,
      virus_graph_node_kmer_ft, virus_graph_node_pssm_ft, ai, vi,
      *weights)
    return out
